# baseline (speedup 1.0000x reference)
import functools

import jax
import jax.numpy as jnp
from jax import lax
from jax.experimental import pallas as pl
from jax.experimental.pallas import tpu as pltpu
from jax.experimental.pallas import tpu_sc as plsc

N_NODES = 10000
N_EDGES = 160000
SEQ_LEN = 6
DIM = 3
LATENT = 128
RADIUS = 0.015

_NC = 2
_NS = 16
_NW = _NC * _NS
_GC = 128
_NCH = N_EDGES // _GC
_CPW = 40
_NCHP = _NW * _CPW
_EPAD = _NCHP * _GC
_ZROWS = 200
_NZCH = N_NODES // _ZROWS
_ZB = 40
_SD = 2
_NACC = N_NODES + 8

_f32 = jnp.float32
_i32 = jnp.int32


def _sc_mesh():
    return plsc.VectorSubcoreMesh(
        core_axis_name="c", subcore_axis_name="s",
        num_cores=_NC, num_subcores=_NS)


_GDEPTH = 4


@functools.lru_cache(maxsize=None)
def _make_gather_add():
    d = LATENT

    @functools.partial(
        pl.kernel,
        mesh=_sc_mesh(),
        out_type=jax.ShapeDtypeStruct((_EPAD, d), _f32),
        scratch_types=[
            pltpu.VMEM((_GC,), _i32),
            pltpu.VMEM((_GC,), _i32),
            pltpu.VMEM((_GC, d), _f32),
            pltpu.SemaphoreType.DMA,
        ],
    )
    def gather_add(ta, tb, s1d, r1d, out, isv, irv, rows, sem):
        wid = lax.axis_index("s") * _NC + lax.axis_index("c")

        def body(i, carry):
            off = (i * _NW + wid) * _GC
            pltpu.sync_copy(s1d.at[pl.ds(off, _GC)], isv)
            pltpu.sync_copy(r1d.at[pl.ds(off, _GC)], irv)
            pltpu.async_copy(ta.at[isv], rows, sem).wait()
            pltpu.async_copy(tb.at[irv], rows, sem, add=True).wait()
            pltpu.sync_copy(rows, out.at[pl.ds(off, _GC)])
            return carry

        lax.fori_loop(0, _CPW, body, 0, unroll=False)

    return gather_add


@functools.lru_cache(maxsize=None)
def _make_segment_sum():
    d = LATENT

    @functools.partial(
        pl.kernel,
        mesh=_sc_mesh(),
        out_type=jax.ShapeDtypeStruct((_NC, N_NODES, d), _f32),
        scratch_types=[pltpu.VMEM((_GC,), _i32)] * _SD
        + [pltpu.VMEM((_GC, d), _f32)] * _SD + [
            pltpu.VMEM((_ZB, d), _f32),
            pltpu.VMEM_SHARED((_NACC, d), _f32),
            pltpu.SemaphoreType.DMA,
            pltpu.SemaphoreType.DMA,
            pltpu.SemaphoreType.DMA,
        ],
    )
    def seg_sum(edges_h, r1d, out, *rest):
        idxs = rest[:_SD]
        bufs = rest[_SD:2 * _SD]
        zbuf, acc, sem_i, sem_l, sem_s = rest[2 * _SD:]
        cid = lax.axis_index("c")
        sid = lax.axis_index("s")
        wid = sid * _NC + cid
        c0 = wid * _CPW

        zv = jnp.zeros((16,), _f32)

        def zrow(i, carry):
            for j in range(d // 16):
                zbuf[i, pl.ds(j * 16, 16)] = zv
            return carry

        lax.fori_loop(0, _ZB, zrow, 0, unroll=False)

        n_zmy = (N_NODES // _ZB - 1 - sid) // _NS + 1

        def zinit(i, carry):
            pltpu.sync_copy(zbuf, acc.at[pl.ds((i * _NS + sid) * _ZB, _ZB)])
            return carry

        lax.fori_loop(0, n_zmy, zinit, 0, unroll=False)
        plsc.subcore_barrier()

        def grp(j, carry):
            cbase = c0 + _SD * j
            di = [pltpu.async_copy(
                r1d.at[pl.ds((cbase + b) * _GC, _GC)], idxs[b],
                sem_i) for b in range(_SD)]
            dl = [pltpu.async_copy(
                edges_h.at[pl.ds(jnp.minimum(cbase + b, _NCH - 1) * _GC,
                                 _GC)], bufs[b],
                sem_l) for b in range(_SD)]
            for x in di:
                x.wait()
            for x in dl:
                x.wait()
            ds = [pltpu.async_copy(bufs[b], acc.at[idxs[b]], sem_s,
                                   add=True) for b in range(_SD)]
            for x in ds:
                x.wait()
            return carry

        lax.fori_loop(0, _CPW // _SD, grp, 0, unroll=False)
        plsc.subcore_barrier()

        n_wmy = (_NZCH - 1 - sid) // _NS + 1

        def wout(i, carry):
            off = (i * _NS + sid) * _ZROWS
            pltpu.sync_copy(acc.at[pl.ds(off, _ZROWS)],
                            out.at[cid, pl.ds(off, _ZROWS)])
            return carry

        lax.fori_loop(0, n_wmy, wout, 0, unroll=False)

    return seg_sum


_BE = 2000
_BN = 2000


def _ln(h, s, o):
    m = jnp.mean(h, axis=-1, keepdims=True)
    v = jnp.mean((h - m) * (h - m), axis=-1, keepdims=True)
    return (h - m) * lax.rsqrt(v + 1e-5) * s + o


def _dot(x, w):
    return jnp.dot(x.astype(jnp.bfloat16), w, preferred_element_type=_f32)


def _row2(b, w=LATENT):
    return pl.BlockSpec((b, w), lambda i: (i, 0))


def _fix(a):
    return pl.BlockSpec(a.shape, lambda i: tuple(0 for _ in a.shape))


def _wcast(p):
    ws = [w.astype(jnp.bfloat16) for w in p["w"]]
    bs = [b.reshape(1, -1) for b in p["b"]]
    extra = []
    if "ln_s" in p:
        extra = [p["ln_s"].reshape(1, -1), p["ln_o"].reshape(1, -1)]
    return ws, bs, extra


def _prep_tables(last, wprep):
    def body(x_ref, w_ref, ta_out, tb_out):
        t = jnp.dot(x_ref[...], w_ref[...], preferred_element_type=_f32)
        ta_out[...] = -t
        tb_out[...] = t

    shp = jax.ShapeDtypeStruct((N_NODES, LATENT), _f32)
    return pl.pallas_call(
        body, grid=(N_NODES // _BN,),
        in_specs=[_row2(_BN, DIM), _fix(wprep)],
        out_specs=[_row2(_BN)] * 2,
        out_shape=[shp, shp],
    )(last, wprep)


def _edge_encoder(g, p):
    ws, bs, (lns, lno) = _wcast(p)
    w1 = p["w"][0]
    m = jnp.eye(LATENT, dtype=_f32).at[:DIM, :DIM].set(w1[:DIM, :DIM])
    m = m.astype(jnp.bfloat16)
    w4 = w1[DIM].reshape(1, LATENT)
    msk = jnp.zeros((1, LATENT), _f32).at[0, :DIM].set(1.0)

    def body(x_ref, m_, w4_, msk_, w2_, w3_, b1_, b2_, b3_, s_, o_, out):
        x = x_ref[...]
        xm = x * msk_[...]
        dist = jnp.sqrt(jnp.sum(xm * xm, axis=-1, keepdims=True))
        h = _dot(x, m_[...]) + dist * w4_[...] + b1_[...]
        h = jnp.maximum(h, 0.0)
        h = jnp.maximum(_dot(h, w2_[...]) + b2_[...], 0.0)
        h = _dot(h, w3_[...]) + b3_[...]
        out[...] = _ln(h, s_[...], o_[...])

    args = (g, m, w4, msk, ws[1], ws[2], bs[0], bs[1], bs[2], lns, lno)
    return pl.pallas_call(
        body, grid=(N_EDGES // _BE,),
        in_specs=[_row2(_BE)] + [_fix(a) for a in args[1:]],
        out_specs=_row2(_BE),
        out_shape=jax.ShapeDtypeStruct((N_EDGES, LATENT), _f32),
    )(*args)


def _node_encoder(wp_hi, wp_lo, p, wb, wc):
    ws, bs, (lns, lno) = _wcast(p)

    def body(a_ref, b_ref, w1_, w2_, w3_, b1_, b2_, b3_, s_, o_, wb_, wc_,
             n_out, pb_out, pc_out):
        vel = (a_ref[...] - b_ref[...]) * (1.0 / RADIUS)
        h = _dot(vel, w1_[...]) + b1_[...]
        h = jnp.maximum(h, 0.0)
        h = jnp.maximum(_dot(h, w2_[...]) + b2_[...], 0.0)
        h = _dot(h, w3_[...]) + b3_[...]
        n = _ln(h, s_[...], o_[...])
        n_out[...] = n
        pb_out[...] = _dot(n, wb_[...])
        pc_out[...] = _dot(n, wc_[...])

    args = (wp_hi, wp_lo, ws[0], ws[1], ws[2], bs[0], bs[1], bs[2], lns, lno,
            wb, wc)
    shp = jax.ShapeDtypeStruct((N_NODES, LATENT), _f32)
    return pl.pallas_call(
        body, grid=(N_NODES // _BN,),
        in_specs=[_row2(_BN, (SEQ_LEN - 1) * DIM)] * 2
        + [_fix(a) for a in args[2:]],
        out_specs=[_row2(_BN)] * 3,
        out_shape=[shp, shp, shp],
    )(*args)


def _edge_step(edges, gsum, p):
    ws, bs, (lns, lno) = _wcast(p)
    wa = ws[0][:LATENT]

    def body(e_ref, g_ref, wa_, w2_, w3_, b1_, b2_, b3_, s_, o_, out):
        x = e_ref[...]
        h = _dot(x, wa_[...]) + g_ref[...] + b1_[...]
        h = jnp.maximum(h, 0.0)
        h = jnp.maximum(_dot(h, w2_[...]) + b2_[...], 0.0)
        h = _dot(h, w3_[...]) + b3_[...]
        out[...] = x + _ln(h, s_[...], o_[...])

    args = (edges, gsum, wa, ws[1], ws[2], bs[0], bs[1], bs[2], lns, lno)
    return pl.pallas_call(
        body, grid=(N_EDGES // _BE,),
        in_specs=[_row2(_BE), _row2(_BE)] + [_fix(a) for a in args[2:]],
        out_specs=_row2(_BE),
        out_shape=jax.ShapeDtypeStruct((N_EDGES, LATENT), _f32),
    )(*args)


def _node_step(nodes, agg0, agg1, p, wb, wc):
    ws, bs, (lns, lno) = _wcast(p)
    wna = ws[0][:LATENT]
    wnb = ws[0][LATENT:]
    with_proj = wb is not None

    def body(n_ref, a0_ref, a1_ref, wna_, wnb_, w2_, w3_, b1_, b2_, b3_,
             s_, o_, *rest):
        if with_proj:
            wb_, wc_, n_out, pb_out, pc_out = rest
        else:
            wb_ = wc_ = pb_out = pc_out = None
            (n_out,) = rest
        x = n_ref[...]
        agg = a0_ref[...] + a1_ref[...]
        h = _dot(x, wna_[...]) + _dot(agg, wnb_[...]) + b1_[...]
        h = jnp.maximum(h, 0.0)
        h = jnp.maximum(_dot(h, w2_[...]) + b2_[...], 0.0)
        h = _dot(h, w3_[...]) + b3_[...]
        n = x + _ln(h, s_[...], o_[...])
        n_out[...] = n
        if with_proj:
            pb_out[...] = _dot(n, wb_[...])
            pc_out[...] = _dot(n, wc_[...])

    args = [nodes, agg0, agg1, wna, wnb, ws[1], ws[2], bs[0], bs[1], bs[2],
            lns, lno]
    if with_proj:
        args += [wb, wc]
    shp = jax.ShapeDtypeStruct((N_NODES, LATENT), _f32)
    n_out = 3 if with_proj else 1
    res = pl.pallas_call(
        body, grid=(N_NODES // _BN,),
        in_specs=[_row2(_BN)] * 3 + [_fix(a) for a in args[3:]],
        out_specs=[_row2(_BN)] * n_out,
        out_shape=[shp] * n_out,
    )(*args)
    return res if with_proj else (res[0], None, None)


def _decoder(nodes, intg, p, stdp, meanp):
    ws = [w.astype(jnp.bfloat16) for w in p["w"]]
    w3p = jnp.zeros((LATENT, LATENT), jnp.bfloat16).at[:, :DIM].set(ws[2])
    bs = [b.reshape(1, -1) for b in p["b"]]
    b3p = jnp.zeros((1, LATENT), _f32).at[0, :DIM].set(p["b"][2])

    def body(n_ref, i_ref, w1_, w2_, w3_, b1_, b2_, b3_, st_, mn_, out):
        h = jnp.maximum(_dot(n_ref[...], w1_[...]) + b1_[...], 0.0)
        h = jnp.maximum(_dot(h, w2_[...]) + b2_[...], 0.0)
        h = _dot(h, w3_[...]) + b3_[...]
        out[...] = i_ref[...] + h * st_[...] + mn_[...]

    args = (nodes, intg, ws[0], ws[1], w3p, bs[0], bs[1], b3p, stdp, meanp)
    return pl.pallas_call(
        body, grid=(N_NODES // _BN,),
        in_specs=[_row2(_BN), _row2(_BN)] + [_fix(a) for a in args[2:]],
        out_specs=_row2(_BN),
        out_shape=jax.ShapeDtypeStruct((N_NODES, LATENT), _f32),
    )(*args)


def kernel(world_position, params, senders, receivers):
    s32 = jnp.pad(senders.astype(_i32), (0, _EPAD - N_EDGES))
    r32 = jnp.pad(receivers.astype(_i32), (0, _EPAD - N_EDGES))
    r32d = jnp.pad(receivers.astype(_i32), (0, _EPAD - N_EDGES),
                   constant_values=N_NODES)
    wp = world_position.reshape(N_NODES, SEQ_LEN * DIM)
    last = wp[:, (SEQ_LEN - 1) * DIM:]
    prev = wp[:, (SEQ_LEN - 2) * DIM:(SEQ_LEN - 1) * DIM]

    w1e = params["enc_edge"]["w"][0]
    wprep = jnp.concatenate(
        [jnp.eye(DIM, dtype=_f32), w1e[:DIM, DIM:]], axis=1) * (1.0 / RADIUS)
    ta, tb = _prep_tables(last, wprep)
    gather_add = _make_gather_add()
    grel = gather_add(ta, tb, s32, r32)

    edges = _edge_encoder(grel, params["enc_edge"])

    pe0 = params["proc"][0]["edge"]
    wb0 = pe0["w"][0][LATENT:2 * LATENT].astype(jnp.bfloat16)
    wc0 = pe0["w"][0][2 * LATENT:].astype(jnp.bfloat16)
    nodes, nsb, nsc = _node_encoder(
        wp[:, DIM:], wp[:, :(SEQ_LEN - 1) * DIM],
        params["enc_node"], wb0, wc0)

    seg_sum = _make_segment_sum()
    n_steps = len(params["proc"])
    for i in range(n_steps):
        gsum = gather_add(nsb, nsc, s32, r32)
        edges = _edge_step(edges, gsum, params["proc"][i]["edge"])
        agg = seg_sum(edges, r32d)
        if i + 1 < n_steps:
            pe = params["proc"][i + 1]["edge"]
            wb = pe["w"][0][LATENT:2 * LATENT].astype(jnp.bfloat16)
            wc = pe["w"][0][2 * LATENT:].astype(jnp.bfloat16)
        else:
            wb = wc = None
        nodes, nsb, nsc = _node_step(
            nodes, agg[0], agg[1], params["proc"][i]["node"], wb, wc)

    intg = jnp.zeros((N_NODES, LATENT), _f32).at[:, :DIM].set(2.0 * last - prev)
    stdp = jnp.zeros((1, LATENT), _f32).at[0, :DIM].set(params["target_std"])
    meanp = jnp.zeros((1, LATENT), _f32).at[0, :DIM].set(params["target_mean"])
    out = _decoder(nodes, intg, params["dec"], stdp, meanp)
    return out[:, :DIM]


# speedup vs baseline: 1.5734x; 1.5734x over previous
import functools

import jax
import jax.numpy as jnp
from jax import lax
from jax.experimental import pallas as pl
from jax.experimental.pallas import tpu as pltpu
from jax.experimental.pallas import tpu_sc as plsc

N_NODES = 10000
N_EDGES = 160000
SEQ_LEN = 6
DIM = 3
LATENT = 128
RADIUS = 0.015

_NC = 2
_NS = 16
_NW = _NC * _NS
_GC = 128
_NCH = N_EDGES // _GC
_CPW = 40
_NCHP = _NW * _CPW
_EPAD = _NCHP * _GC
_ZROWS = 200
_NZCH = N_NODES // _ZROWS
_ZB = 40
_SD = 2
_NACC = N_NODES + 8

_f32 = jnp.float32
_i32 = jnp.int32


def _sc_mesh():
    return plsc.VectorSubcoreMesh(
        core_axis_name="c", subcore_axis_name="s",
        num_cores=_NC, num_subcores=_NS)


_GDEPTH = 4


@functools.lru_cache(maxsize=None)
def _make_gather_add():
    d = LATENT

    @functools.partial(
        pl.kernel,
        mesh=_sc_mesh(),
        out_type=jax.ShapeDtypeStruct((N_EDGES, d), _f32),
        scratch_types=[
            pltpu.VMEM((_GC,), _i32),
            pltpu.VMEM((_GC,), _i32),
            pltpu.VMEM((_GC, d), _f32),
            pltpu.SemaphoreType.DMA,
        ],
    )
    def gather_add(ta, tb, s1d, r1d, out, isv, irv, rows, sem):
        wid = lax.axis_index("s") * _NC + lax.axis_index("c")
        n_my = (_NCH - 1 - wid) // _NW + 1

        def body(i, carry):
            off = (i * _NW + wid) * _GC
            pltpu.sync_copy(s1d.at[pl.ds(off, _GC)], isv)
            pltpu.sync_copy(r1d.at[pl.ds(off, _GC)], irv)
            pltpu.async_copy(ta.at[isv], rows, sem).wait()
            pltpu.async_copy(tb.at[irv], rows, sem, add=True).wait()
            pltpu.sync_copy(rows, out.at[pl.ds(off, _GC)])
            return carry

        lax.fori_loop(0, n_my, body, 0, unroll=False)

    return gather_add


@functools.lru_cache(maxsize=None)
def _make_segment_sum():
    d = LATENT

    @functools.partial(
        pl.kernel,
        mesh=_sc_mesh(),
        out_type=jax.ShapeDtypeStruct((_NC, N_NODES, d), _f32),
        scratch_types=[pltpu.VMEM((_GC,), _i32)] * _SD
        + [pltpu.VMEM((_GC, d), _f32)] * _SD + [
            pltpu.VMEM((_ZB, d), _f32),
            pltpu.VMEM_SHARED((_NACC, d), _f32),
            pltpu.SemaphoreType.DMA,
            pltpu.SemaphoreType.DMA,
            pltpu.SemaphoreType.DMA,
        ],
    )
    def seg_sum(edges_h, r1d, out, *rest):
        idxs = rest[:_SD]
        bufs = rest[_SD:2 * _SD]
        zbuf, acc, sem_i, sem_l, sem_s = rest[2 * _SD:]
        cid = lax.axis_index("c")
        sid = lax.axis_index("s")
        wid = sid * _NC + cid
        c0 = wid * _CPW

        zv = jnp.zeros((16,), _f32)

        def zrow(i, carry):
            for j in range(d // 16):
                zbuf[i, pl.ds(j * 16, 16)] = zv
            return carry

        lax.fori_loop(0, _ZB, zrow, 0, unroll=False)

        n_zmy = (N_NODES // _ZB - 1 - sid) // _NS + 1

        def zinit(i, carry):
            pltpu.sync_copy(zbuf, acc.at[pl.ds((i * _NS + sid) * _ZB, _ZB)])
            return carry

        lax.fori_loop(0, n_zmy, zinit, 0, unroll=False)
        plsc.subcore_barrier()

        def grp(j, carry):
            cbase = c0 + _SD * j
            di = [pltpu.async_copy(
                r1d.at[pl.ds((cbase + b) * _GC, _GC)], idxs[b],
                sem_i) for b in range(_SD)]
            dl = [pltpu.async_copy(
                edges_h.at[pl.ds(jnp.minimum(cbase + b, _NCH - 1) * _GC,
                                 _GC)], bufs[b],
                sem_l) for b in range(_SD)]
            for x in di:
                x.wait()
            for x in dl:
                x.wait()
            ds = [pltpu.async_copy(bufs[b], acc.at[idxs[b]], sem_s,
                                   add=True) for b in range(_SD)]
            for x in ds:
                x.wait()
            return carry

        lax.fori_loop(0, _CPW // _SD, grp, 0, unroll=False)
        plsc.subcore_barrier()

        n_wmy = (_NZCH - 1 - sid) // _NS + 1

        def wout(i, carry):
            off = (i * _NS + sid) * _ZROWS
            pltpu.sync_copy(acc.at[pl.ds(off, _ZROWS)],
                            out.at[cid, pl.ds(off, _ZROWS)])
            return carry

        lax.fori_loop(0, n_wmy, wout, 0, unroll=False)

    return seg_sum


_BE = 2000
_BN = 2000


def _ln(h, s, o):
    m = jnp.mean(h, axis=-1, keepdims=True)
    v = jnp.mean((h - m) * (h - m), axis=-1, keepdims=True)
    return (h - m) * lax.rsqrt(v + 1e-5) * s + o


def _dot(x, w):
    return jnp.dot(x.astype(jnp.bfloat16), w, preferred_element_type=_f32)


def _row2(b, w=LATENT):
    return pl.BlockSpec((b, w), lambda i: (i, 0))


def _fix(a):
    return pl.BlockSpec(a.shape, lambda i: tuple(0 for _ in a.shape))


def _wcast(p):
    ws = [w.astype(jnp.bfloat16) for w in p["w"]]
    bs = [b.reshape(1, -1) for b in p["b"]]
    extra = []
    if "ln_s" in p:
        extra = [p["ln_s"].reshape(1, -1), p["ln_o"].reshape(1, -1)]
    return ws, bs, extra


def _prep_tables(last, wprep):
    def body(x_ref, w_ref, ta_out, tb_out):
        t = jnp.dot(x_ref[...], w_ref[...], preferred_element_type=_f32)
        ta_out[...] = -t
        tb_out[...] = t

    shp = jax.ShapeDtypeStruct((N_NODES, LATENT), _f32)
    return pl.pallas_call(
        body, grid=(N_NODES // _BN,),
        in_specs=[_row2(_BN, DIM), _fix(wprep)],
        out_specs=[_row2(_BN)] * 2,
        out_shape=[shp, shp],
    )(last, wprep)


def _edge_encoder(g, p):
    ws, bs, (lns, lno) = _wcast(p)
    w1 = p["w"][0]
    m = jnp.eye(LATENT, dtype=_f32).at[:DIM, :DIM].set(w1[:DIM, :DIM])
    m = m.astype(jnp.bfloat16)
    w4 = w1[DIM].reshape(1, LATENT)
    msk = jnp.zeros((1, LATENT), _f32).at[0, :DIM].set(1.0)

    def body(x_ref, m_, w4_, msk_, w2_, w3_, b1_, b2_, b3_, s_, o_, out):
        x = x_ref[...]
        xm = x * msk_[...]
        dist = jnp.sqrt(jnp.sum(xm * xm, axis=-1, keepdims=True))
        h = _dot(x, m_[...]) + dist * w4_[...] + b1_[...]
        h = jnp.maximum(h, 0.0)
        h = jnp.maximum(_dot(h, w2_[...]) + b2_[...], 0.0)
        h = _dot(h, w3_[...]) + b3_[...]
        out[...] = _ln(h, s_[...], o_[...])

    args = (g, m, w4, msk, ws[1], ws[2], bs[0], bs[1], bs[2], lns, lno)
    return pl.pallas_call(
        body, grid=(N_EDGES // _BE,),
        in_specs=[_row2(_BE)] + [_fix(a) for a in args[1:]],
        out_specs=_row2(_BE),
        out_shape=jax.ShapeDtypeStruct((N_EDGES, LATENT), _f32),
    )(*args)


def _node_encoder(wp_hi, wp_lo, p, wb, wc):
    ws, bs, (lns, lno) = _wcast(p)

    def body(a_ref, b_ref, w1_, w2_, w3_, b1_, b2_, b3_, s_, o_, wb_, wc_,
             n_out, pb_out, pc_out):
        vel = (a_ref[...] - b_ref[...]) * (1.0 / RADIUS)
        h = _dot(vel, w1_[...]) + b1_[...]
        h = jnp.maximum(h, 0.0)
        h = jnp.maximum(_dot(h, w2_[...]) + b2_[...], 0.0)
        h = _dot(h, w3_[...]) + b3_[...]
        n = _ln(h, s_[...], o_[...])
        n_out[...] = n
        pb_out[...] = _dot(n, wb_[...])
        pc_out[...] = _dot(n, wc_[...])

    args = (wp_hi, wp_lo, ws[0], ws[1], ws[2], bs[0], bs[1], bs[2], lns, lno,
            wb, wc)
    shp = jax.ShapeDtypeStruct((N_NODES, LATENT), _f32)
    return pl.pallas_call(
        body, grid=(N_NODES // _BN,),
        in_specs=[_row2(_BN, (SEQ_LEN - 1) * DIM)] * 2
        + [_fix(a) for a in args[2:]],
        out_specs=[_row2(_BN)] * 3,
        out_shape=[shp, shp, shp],
    )(*args)


def _edge_step(edges, gsum, p):
    ws, bs, (lns, lno) = _wcast(p)
    wa = ws[0][:LATENT]

    def body(e_ref, g_ref, wa_, w2_, w3_, b1_, b2_, b3_, s_, o_, out):
        x = e_ref[...]
        h = _dot(x, wa_[...]) + g_ref[...] + b1_[...]
        h = jnp.maximum(h, 0.0)
        h = jnp.maximum(_dot(h, w2_[...]) + b2_[...], 0.0)
        h = _dot(h, w3_[...]) + b3_[...]
        out[...] = x + _ln(h, s_[...], o_[...])

    args = (edges, gsum, wa, ws[1], ws[2], bs[0], bs[1], bs[2], lns, lno)
    return pl.pallas_call(
        body, grid=(N_EDGES // _BE,),
        in_specs=[_row2(_BE), _row2(_BE)] + [_fix(a) for a in args[2:]],
        out_specs=_row2(_BE),
        out_shape=jax.ShapeDtypeStruct((N_EDGES, LATENT), _f32),
    )(*args)


def _node_step(nodes, agg0, agg1, p, wb, wc):
    ws, bs, (lns, lno) = _wcast(p)
    wna = ws[0][:LATENT]
    wnb = ws[0][LATENT:]
    with_proj = wb is not None

    def body(n_ref, a0_ref, a1_ref, wna_, wnb_, w2_, w3_, b1_, b2_, b3_,
             s_, o_, *rest):
        if with_proj:
            wb_, wc_, n_out, pb_out, pc_out = rest
        else:
            wb_ = wc_ = pb_out = pc_out = None
            (n_out,) = rest
        x = n_ref[...]
        agg = a0_ref[...] + a1_ref[...]
        h = _dot(x, wna_[...]) + _dot(agg, wnb_[...]) + b1_[...]
        h = jnp.maximum(h, 0.0)
        h = jnp.maximum(_dot(h, w2_[...]) + b2_[...], 0.0)
        h = _dot(h, w3_[...]) + b3_[...]
        n = x + _ln(h, s_[...], o_[...])
        n_out[...] = n
        if with_proj:
            pb_out[...] = _dot(n, wb_[...])
            pc_out[...] = _dot(n, wc_[...])

    args = [nodes, agg0, agg1, wna, wnb, ws[1], ws[2], bs[0], bs[1], bs[2],
            lns, lno]
    if with_proj:
        args += [wb, wc]
    shp = jax.ShapeDtypeStruct((N_NODES, LATENT), _f32)
    n_out = 3 if with_proj else 1
    res = pl.pallas_call(
        body, grid=(N_NODES // _BN,),
        in_specs=[_row2(_BN)] * 3 + [_fix(a) for a in args[3:]],
        out_specs=[_row2(_BN)] * n_out,
        out_shape=[shp] * n_out,
    )(*args)
    return res if with_proj else (res[0], None, None)


def _decoder(nodes, intg, p, stdp, meanp):
    ws = [w.astype(jnp.bfloat16) for w in p["w"]]
    w3p = jnp.zeros((LATENT, LATENT), jnp.bfloat16).at[:, :DIM].set(ws[2])
    bs = [b.reshape(1, -1) for b in p["b"]]
    b3p = jnp.zeros((1, LATENT), _f32).at[0, :DIM].set(p["b"][2])

    def body(n_ref, i_ref, w1_, w2_, w3_, b1_, b2_, b3_, st_, mn_, out):
        h = jnp.maximum(_dot(n_ref[...], w1_[...]) + b1_[...], 0.0)
        h = jnp.maximum(_dot(h, w2_[...]) + b2_[...], 0.0)
        h = _dot(h, w3_[...]) + b3_[...]
        out[...] = i_ref[...] + h * st_[...] + mn_[...]

    args = (nodes, intg, ws[0], ws[1], w3p, bs[0], bs[1], b3p, stdp, meanp)
    return pl.pallas_call(
        body, grid=(N_NODES // _BN,),
        in_specs=[_row2(_BN), _row2(_BN)] + [_fix(a) for a in args[2:]],
        out_specs=_row2(_BN),
        out_shape=jax.ShapeDtypeStruct((N_NODES, LATENT), _f32),
    )(*args)


def kernel(world_position, params, senders, receivers):
    s32 = senders.astype(_i32)
    r32 = receivers.astype(_i32)
    r32d = jnp.pad(r32, (0, _EPAD - N_EDGES), constant_values=N_NODES)
    wp = world_position.reshape(N_NODES, SEQ_LEN * DIM)
    last = wp[:, (SEQ_LEN - 1) * DIM:]
    prev = wp[:, (SEQ_LEN - 2) * DIM:(SEQ_LEN - 1) * DIM]

    w1e = params["enc_edge"]["w"][0]
    wprep = jnp.concatenate(
        [jnp.eye(DIM, dtype=_f32), w1e[:DIM, DIM:]], axis=1) * (1.0 / RADIUS)
    ta, tb = _prep_tables(last, wprep)
    gather_add = _make_gather_add()
    grel = gather_add(ta, tb, s32, r32)

    edges = _edge_encoder(grel, params["enc_edge"])

    pe0 = params["proc"][0]["edge"]
    wb0 = pe0["w"][0][LATENT:2 * LATENT].astype(jnp.bfloat16)
    wc0 = pe0["w"][0][2 * LATENT:].astype(jnp.bfloat16)
    nodes, nsb, nsc = _node_encoder(
        wp[:, DIM:], wp[:, :(SEQ_LEN - 1) * DIM],
        params["enc_node"], wb0, wc0)

    seg_sum = _make_segment_sum()
    n_steps = len(params["proc"])
    for i in range(n_steps):
        gsum = gather_add(nsb, nsc, s32, r32)
        edges = _edge_step(edges, gsum, params["proc"][i]["edge"])
        agg = seg_sum(edges, r32d)
        if i + 1 < n_steps:
            pe = params["proc"][i + 1]["edge"]
            wb = pe["w"][0][LATENT:2 * LATENT].astype(jnp.bfloat16)
            wc = pe["w"][0][2 * LATENT:].astype(jnp.bfloat16)
        else:
            wb = wc = None
        nodes, nsb, nsc = _node_step(
            nodes, agg[0], agg[1], params["proc"][i]["node"], wb, wc)

    intg = jnp.zeros((N_NODES, LATENT), _f32).at[:, :DIM].set(2.0 * last - prev)
    stdp = jnp.zeros((1, LATENT), _f32).at[0, :DIM].set(params["target_std"])
    meanp = jnp.zeros((1, LATENT), _f32).at[0, :DIM].set(params["target_mean"])
    out = _decoder(nodes, intg, params["dec"], stdp, meanp)
    return out[:, :DIM]


# speedup vs baseline: 1.8554x; 1.1792x over previous
import functools

import jax
import jax.numpy as jnp
from jax import lax
from jax.experimental import pallas as pl
from jax.experimental.pallas import tpu as pltpu
from jax.experimental.pallas import tpu_sc as plsc

N_NODES = 10000
N_EDGES = 160000
SEQ_LEN = 6
DIM = 3
LATENT = 128
RADIUS = 0.015

_NC = 2
_NS = 16
_NW = _NC * _NS
_GC = 128
_NCH = N_EDGES // _GC
_CPW = 40
_NCHP = _NW * _CPW
_EPAD = _NCHP * _GC
_ZROWS = 200
_NZCH = N_NODES // _ZROWS
_ZB = 40
_SD = 2
_NACC = N_NODES + 8

_f32 = jnp.float32
_i32 = jnp.int32


def _sc_mesh():
    return plsc.VectorSubcoreMesh(
        core_axis_name="c", subcore_axis_name="s",
        num_cores=_NC, num_subcores=_NS)


_GDEPTH = 4


@functools.lru_cache(maxsize=None)
def _make_gather_add(ne=N_EDGES):
    d = LATENT
    nch = ne // _GC

    @functools.partial(
        pl.kernel,
        mesh=_sc_mesh(),
        out_type=jax.ShapeDtypeStruct((ne, d), _f32),
        scratch_types=[
            pltpu.VMEM((_GC,), _i32),
            pltpu.VMEM((_GC,), _i32),
            pltpu.VMEM((_GC, d), _f32),
            pltpu.SemaphoreType.DMA,
        ],
    )
    def gather_add(ta, tb, s1d, r1d, out, isv, irv, rows, sem):
        wid = lax.axis_index("s") * _NC + lax.axis_index("c")
        n_my = (nch - 1 - wid) // _NW + 1

        def body(i, carry):
            off = (i * _NW + wid) * _GC
            pltpu.sync_copy(s1d.at[pl.ds(off, _GC)], isv)
            pltpu.sync_copy(r1d.at[pl.ds(off, _GC)], irv)
            pltpu.async_copy(ta.at[isv], rows, sem).wait()
            pltpu.async_copy(tb.at[irv], rows, sem, add=True).wait()
            pltpu.sync_copy(rows, out.at[pl.ds(off, _GC)])
            return carry

        lax.fori_loop(0, n_my, body, 0, unroll=False)

    return gather_add


@functools.lru_cache(maxsize=None)
def _make_segment_sum(ne=N_EDGES):
    d = LATENT
    nch = ne // _GC
    cpw = -(-nch // _NW)

    @functools.partial(
        pl.kernel,
        mesh=_sc_mesh(),
        out_type=jax.ShapeDtypeStruct((_NC, N_NODES, d), _f32),
        scratch_types=[pltpu.VMEM((_GC,), _i32)] * _SD
        + [pltpu.VMEM((_GC, d), _f32)] * _SD + [
            pltpu.VMEM((_ZB, d), _f32),
            pltpu.VMEM_SHARED((_NACC, d), _f32),
            pltpu.SemaphoreType.DMA,
            pltpu.SemaphoreType.DMA,
            pltpu.SemaphoreType.DMA,
        ],
    )
    def seg_sum(edges_h, r1d, out, *rest):
        idxs = rest[:_SD]
        bufs = rest[_SD:2 * _SD]
        zbuf, acc, sem_i, sem_l, sem_s = rest[2 * _SD:]
        cid = lax.axis_index("c")
        sid = lax.axis_index("s")
        wid = sid * _NC + cid
        c0 = wid * cpw

        zv = jnp.zeros((16,), _f32)

        def zrow(i, carry):
            for j in range(d // 16):
                zbuf[i, pl.ds(j * 16, 16)] = zv
            return carry

        lax.fori_loop(0, _ZB, zrow, 0, unroll=False)

        n_zmy = (N_NODES // _ZB - 1 - sid) // _NS + 1

        def zinit(i, carry):
            pltpu.sync_copy(zbuf, acc.at[pl.ds((i * _NS + sid) * _ZB, _ZB)])
            return carry

        lax.fori_loop(0, n_zmy, zinit, 0, unroll=False)
        plsc.subcore_barrier()

        def grp(j, carry):
            cbase = c0 + _SD * j
            di = [pltpu.async_copy(
                r1d.at[pl.ds((cbase + b) * _GC, _GC)], idxs[b],
                sem_i) for b in range(_SD)]
            dl = [pltpu.async_copy(
                edges_h.at[pl.ds(jnp.minimum(cbase + b, nch - 1) * _GC,
                                 _GC)], bufs[b],
                sem_l) for b in range(_SD)]
            for x in di:
                x.wait()
            for x in dl:
                x.wait()
            ds = [pltpu.async_copy(bufs[b], acc.at[idxs[b]], sem_s,
                                   add=True) for b in range(_SD)]
            for x in ds:
                x.wait()
            return carry

        lax.fori_loop(0, cpw // _SD, grp, 0, unroll=False)
        plsc.subcore_barrier()

        n_wmy = (_NZCH - 1 - sid) // _NS + 1

        def wout(i, carry):
            off = (i * _NS + sid) * _ZROWS
            pltpu.sync_copy(acc.at[pl.ds(off, _ZROWS)],
                            out.at[cid, pl.ds(off, _ZROWS)])
            return carry

        lax.fori_loop(0, n_wmy, wout, 0, unroll=False)

    return seg_sum


_BE = 2000
_BN = 2000


def _ln(h, s, o):
    m = jnp.mean(h, axis=-1, keepdims=True)
    v = jnp.mean((h - m) * (h - m), axis=-1, keepdims=True)
    return (h - m) * lax.rsqrt(v + 1e-5) * s + o


def _dot(x, w):
    return jnp.dot(x.astype(jnp.bfloat16), w, preferred_element_type=_f32)


def _row2(b, w=LATENT):
    return pl.BlockSpec((b, w), lambda i: (i, 0))


def _fix(a):
    return pl.BlockSpec(a.shape, lambda i: tuple(0 for _ in a.shape))


def _wcast(p):
    ws = [w.astype(jnp.bfloat16) for w in p["w"]]
    bs = [b.reshape(1, -1) for b in p["b"]]
    extra = []
    if "ln_s" in p:
        extra = [p["ln_s"].reshape(1, -1), p["ln_o"].reshape(1, -1)]
    return ws, bs, extra


def _prep_tables(last, wprep):
    def body(x_ref, w_ref, ta_out, tb_out):
        t = jnp.dot(x_ref[...], w_ref[...], preferred_element_type=_f32)
        ta_out[...] = -t
        tb_out[...] = t

    shp = jax.ShapeDtypeStruct((N_NODES, LATENT), _f32)
    return pl.pallas_call(
        body, grid=(N_NODES // _BN,),
        in_specs=[_row2(_BN, DIM), _fix(wprep)],
        out_specs=[_row2(_BN)] * 2,
        out_shape=[shp, shp],
    )(last, wprep)


def _edge_encoder(g, p):
    ws, bs, (lns, lno) = _wcast(p)
    w1 = p["w"][0]
    m = jnp.eye(LATENT, dtype=_f32).at[:DIM, :DIM].set(w1[:DIM, :DIM])
    m = m.astype(jnp.bfloat16)
    w4 = w1[DIM].reshape(1, LATENT)
    msk = jnp.zeros((1, LATENT), _f32).at[0, :DIM].set(1.0)

    def body(x_ref, m_, w4_, msk_, w2_, w3_, b1_, b2_, b3_, s_, o_, out):
        x = x_ref[...]
        xm = x * msk_[...]
        dist = jnp.sqrt(jnp.sum(xm * xm, axis=-1, keepdims=True))
        h = _dot(x, m_[...]) + dist * w4_[...] + b1_[...]
        h = jnp.maximum(h, 0.0)
        h = jnp.maximum(_dot(h, w2_[...]) + b2_[...], 0.0)
        h = _dot(h, w3_[...]) + b3_[...]
        out[...] = _ln(h, s_[...], o_[...])

    ne = g.shape[0]
    args = (g, m, w4, msk, ws[1], ws[2], bs[0], bs[1], bs[2], lns, lno)
    return pl.pallas_call(
        body, grid=(ne // _BE,),
        in_specs=[_row2(_BE)] + [_fix(a) for a in args[1:]],
        out_specs=_row2(_BE),
        out_shape=jax.ShapeDtypeStruct((ne, LATENT), _f32),
    )(*args)


def _node_encoder(wp_hi, wp_lo, p, wb, wc):
    ws, bs, (lns, lno) = _wcast(p)

    def body(a_ref, b_ref, w1_, w2_, w3_, b1_, b2_, b3_, s_, o_, wb_, wc_,
             n_out, pb_out, pc_out):
        vel = (a_ref[...] - b_ref[...]) * (1.0 / RADIUS)
        h = _dot(vel, w1_[...]) + b1_[...]
        h = jnp.maximum(h, 0.0)
        h = jnp.maximum(_dot(h, w2_[...]) + b2_[...], 0.0)
        h = _dot(h, w3_[...]) + b3_[...]
        n = _ln(h, s_[...], o_[...])
        n_out[...] = n
        pb_out[...] = _dot(n, wb_[...])
        pc_out[...] = _dot(n, wc_[...])

    args = (wp_hi, wp_lo, ws[0], ws[1], ws[2], bs[0], bs[1], bs[2], lns, lno,
            wb, wc)
    shp = jax.ShapeDtypeStruct((N_NODES, LATENT), _f32)
    return pl.pallas_call(
        body, grid=(N_NODES // _BN,),
        in_specs=[_row2(_BN, (SEQ_LEN - 1) * DIM)] * 2
        + [_fix(a) for a in args[2:]],
        out_specs=[_row2(_BN)] * 3,
        out_shape=[shp, shp, shp],
    )(*args)


def _edge_step(edges, gsum, p):
    ws, bs, (lns, lno) = _wcast(p)
    wa = ws[0][:LATENT]

    def body(e_ref, g_ref, wa_, w2_, w3_, b1_, b2_, b3_, s_, o_, out):
        x = e_ref[...]
        h = _dot(x, wa_[...]) + g_ref[...] + b1_[...]
        h = jnp.maximum(h, 0.0)
        h = jnp.maximum(_dot(h, w2_[...]) + b2_[...], 0.0)
        h = _dot(h, w3_[...]) + b3_[...]
        out[...] = x + _ln(h, s_[...], o_[...])

    ne = edges.shape[0]
    args = (edges, gsum, wa, ws[1], ws[2], bs[0], bs[1], bs[2], lns, lno)
    return pl.pallas_call(
        body, grid=(ne // _BE,),
        in_specs=[_row2(_BE), _row2(_BE)] + [_fix(a) for a in args[2:]],
        out_specs=_row2(_BE),
        out_shape=jax.ShapeDtypeStruct((ne, LATENT), _f32),
    )(*args)


def _node_step(nodes, aggs, p, wb, wc):
    ws, bs, (lns, lno) = _wcast(p)
    wna = ws[0][:LATENT]
    wnb = ws[0][LATENT:]
    with_proj = wb is not None
    na = len(aggs)

    def body(n_ref, *rest):
        a_refs = rest[:na]
        wna_, wnb_, w2_, w3_, b1_, b2_, b3_, s_, o_ = rest[na:na + 9]
        rest = rest[na + 9:]
        if with_proj:
            wb_, wc_, n_out, pb_out, pc_out = rest
        else:
            wb_ = wc_ = pb_out = pc_out = None
            (n_out,) = rest
        x = n_ref[...]
        agg = a_refs[0][...]
        for a in a_refs[1:]:
            agg = agg + a[...]
        h = _dot(x, wna_[...]) + _dot(agg, wnb_[...]) + b1_[...]
        h = jnp.maximum(h, 0.0)
        h = jnp.maximum(_dot(h, w2_[...]) + b2_[...], 0.0)
        h = _dot(h, w3_[...]) + b3_[...]
        n = x + _ln(h, s_[...], o_[...])
        n_out[...] = n
        if with_proj:
            pb_out[...] = _dot(n, wb_[...])
            pc_out[...] = _dot(n, wc_[...])

    args = [nodes] + list(aggs) + [wna, wnb, ws[1], ws[2], bs[0], bs[1],
                                   bs[2], lns, lno]
    if with_proj:
        args += [wb, wc]
    shp = jax.ShapeDtypeStruct((N_NODES, LATENT), _f32)
    n_out = 3 if with_proj else 1
    res = pl.pallas_call(
        body, grid=(N_NODES // _BN,),
        in_specs=[_row2(_BN)] * (1 + na) + [_fix(a) for a in args[1 + na:]],
        out_specs=[_row2(_BN)] * n_out,
        out_shape=[shp] * n_out,
    )(*args)
    return res if with_proj else (res[0], None, None)


def _decoder(nodes, intg, p, stdp, meanp):
    ws = [w.astype(jnp.bfloat16) for w in p["w"]]
    w3p = jnp.zeros((LATENT, LATENT), jnp.bfloat16).at[:, :DIM].set(ws[2])
    bs = [b.reshape(1, -1) for b in p["b"]]
    b3p = jnp.zeros((1, LATENT), _f32).at[0, :DIM].set(p["b"][2])

    def body(n_ref, i_ref, w1_, w2_, w3_, b1_, b2_, b3_, st_, mn_, out):
        h = jnp.maximum(_dot(n_ref[...], w1_[...]) + b1_[...], 0.0)
        h = jnp.maximum(_dot(h, w2_[...]) + b2_[...], 0.0)
        h = _dot(h, w3_[...]) + b3_[...]
        out[...] = i_ref[...] + h * st_[...] + mn_[...]

    args = (nodes, intg, ws[0], ws[1], w3p, bs[0], bs[1], b3p, stdp, meanp)
    return pl.pallas_call(
        body, grid=(N_NODES // _BN,),
        in_specs=[_row2(_BN), _row2(_BN)] + [_fix(a) for a in args[2:]],
        out_specs=_row2(_BN),
        out_shape=jax.ShapeDtypeStruct((N_NODES, LATENT), _f32),
    )(*args)


_NHALF = 2
_EH = N_EDGES // _NHALF


def kernel(world_position, params, senders, receivers):
    s32 = senders.astype(_i32)
    r32 = receivers.astype(_i32)
    ncp = _NW * (-(-(_EH // _GC) // _NW)) * _GC
    sh = [s32[h * _EH:(h + 1) * _EH] for h in range(_NHALF)]
    rh = [r32[h * _EH:(h + 1) * _EH] for h in range(_NHALF)]
    rhd = [jnp.pad(r, (0, ncp - _EH), constant_values=N_NODES) for r in rh]
    wp = world_position.reshape(N_NODES, SEQ_LEN * DIM)
    last = wp[:, (SEQ_LEN - 1) * DIM:]
    prev = wp[:, (SEQ_LEN - 2) * DIM:(SEQ_LEN - 1) * DIM]

    w1e = params["enc_edge"]["w"][0]
    wprep = jnp.concatenate(
        [jnp.eye(DIM, dtype=_f32), w1e[:DIM, DIM:]], axis=1) * (1.0 / RADIUS)
    ta, tb = _prep_tables(last, wprep)
    gather_add = _make_gather_add(_EH)
    grel = [gather_add(ta, tb, sh[h], rh[h]) for h in range(_NHALF)]

    edges = [_edge_encoder(g, params["enc_edge"]) for g in grel]

    pe0 = params["proc"][0]["edge"]
    wb0 = pe0["w"][0][LATENT:2 * LATENT].astype(jnp.bfloat16)
    wc0 = pe0["w"][0][2 * LATENT:].astype(jnp.bfloat16)
    nodes, nsb, nsc = _node_encoder(
        wp[:, DIM:], wp[:, :(SEQ_LEN - 1) * DIM],
        params["enc_node"], wb0, wc0)

    seg_sum = _make_segment_sum(_EH)
    n_steps = len(params["proc"])
    for i in range(n_steps):
        gsum = [gather_add(nsb, nsc, sh[h], rh[h]) for h in range(_NHALF)]
        edges = [_edge_step(edges[h], gsum[h], params["proc"][i]["edge"])
                 for h in range(_NHALF)]
        aggs = []
        for h in range(_NHALF):
            a = seg_sum(edges[h], rhd[h])
            aggs += [a[0], a[1]]
        if i + 1 < n_steps:
            pe = params["proc"][i + 1]["edge"]
            wb = pe["w"][0][LATENT:2 * LATENT].astype(jnp.bfloat16)
            wc = pe["w"][0][2 * LATENT:].astype(jnp.bfloat16)
        else:
            wb = wc = None
        nodes, nsb, nsc = _node_step(
            nodes, aggs, params["proc"][i]["node"], wb, wc)

    intg = jnp.zeros((N_NODES, LATENT), _f32).at[:, :DIM].set(2.0 * last - prev)
    stdp = jnp.zeros((1, LATENT), _f32).at[0, :DIM].set(params["target_std"])
    meanp = jnp.zeros((1, LATENT), _f32).at[0, :DIM].set(params["target_mean"])
    out = _decoder(nodes, intg, params["dec"], stdp, meanp)
    return out[:, :DIM]


# speedup vs baseline: 2.0083x; 1.0824x over previous
import functools

import jax
import jax.numpy as jnp
from jax import lax
from jax.experimental import pallas as pl
from jax.experimental.pallas import tpu as pltpu
from jax.experimental.pallas import tpu_sc as plsc

N_NODES = 10000
N_EDGES = 160000
SEQ_LEN = 6
DIM = 3
LATENT = 128
RADIUS = 0.015

_NC = 2
_NS = 16
_NW = _NC * _NS
_GC = 128
_NCH = N_EDGES // _GC
_CPW = 40
_NCHP = _NW * _CPW
_EPAD = _NCHP * _GC
_ZROWS = 200
_NZCH = N_NODES // _ZROWS
_ZB = 40
_SD = 2
_NACC = N_NODES + 8

_f32 = jnp.float32
_i32 = jnp.int32


def _sc_mesh():
    return plsc.VectorSubcoreMesh(
        core_axis_name="c", subcore_axis_name="s",
        num_cores=_NC, num_subcores=_NS)


_GDEPTH = 4


@functools.lru_cache(maxsize=None)
def _make_gather_add(ne=N_EDGES):
    d = LATENT
    nch = ne // _GC
    cpw = -(-nch // _NW)
    assert cpw % 2 == 0
    nep = _NW * cpw * _GC
    prime = nch * _GC

    @functools.partial(
        pl.kernel,
        mesh=_sc_mesh(),
        out_type=jax.ShapeDtypeStruct((nep, d), _f32),
        scratch_types=[pltpu.VMEM((_GC,), _i32)] * 4
        + [pltpu.VMEM((_GC, d), _f32)] * 2 + [
            pltpu.SemaphoreType.DMA,
            pltpu.SemaphoreType.DMA,
            pltpu.SemaphoreType.DMA,
            pltpu.SemaphoreType.DMA,
        ],
    )
    def gather_add(ta, tb, s1d, r1d, out, isv0, irv0, isv1, irv1,
                   rows0, rows1, sem_i, sem, sem_w0, sem_w1):
        wid = lax.axis_index("s") * _NC + lax.axis_index("c")

        def drain_w(rows, sem_w):
            pltpu.make_async_copy(rows, out.at[pl.ds(prime, _GC)],
                                  sem_w).wait()

        pltpu.async_copy(rows0, out.at[pl.ds(prime, _GC)], sem_w0)
        pltpu.async_copy(rows1, out.at[pl.ds(prime, _GC)], sem_w1)

        def body(j, carry):
            ca = ((2 * j) * _NW + wid) * _GC
            cb = ca + _NW * _GC
            di = [pltpu.async_copy(s1d.at[pl.ds(ca, _GC)], isv0, sem_i),
                  pltpu.async_copy(r1d.at[pl.ds(ca, _GC)], irv0, sem_i),
                  pltpu.async_copy(s1d.at[pl.ds(cb, _GC)], isv1, sem_i),
                  pltpu.async_copy(r1d.at[pl.ds(cb, _GC)], irv1, sem_i)]
            for x in di:
                x.wait()
            drain_w(rows0, sem_w0)
            pltpu.async_copy(ta.at[isv0], rows0, sem).wait()
            pltpu.async_copy(tb.at[irv0], rows0, sem, add=True).wait()
            pltpu.async_copy(rows0, out.at[pl.ds(ca, _GC)], sem_w0)
            drain_w(rows1, sem_w1)
            pltpu.async_copy(ta.at[isv1], rows1, sem).wait()
            pltpu.async_copy(tb.at[irv1], rows1, sem, add=True).wait()
            pltpu.async_copy(rows1, out.at[pl.ds(cb, _GC)], sem_w1)
            return carry

        lax.fori_loop(0, cpw // 2, body, 0, unroll=False)
        drain_w(rows0, sem_w0)
        drain_w(rows1, sem_w1)

    return gather_add


@functools.lru_cache(maxsize=None)
def _make_segment_sum(ne=N_EDGES):
    d = LATENT
    nch = ne // _GC
    cpw = -(-nch // _NW)

    @functools.partial(
        pl.kernel,
        mesh=_sc_mesh(),
        out_type=jax.ShapeDtypeStruct((_NC, N_NODES, d), _f32),
        scratch_types=[pltpu.VMEM((_GC,), _i32)] * _SD
        + [pltpu.VMEM((_GC, d), _f32)] * _SD + [
            pltpu.VMEM((_ZB, d), _f32),
            pltpu.VMEM_SHARED((_NACC, d), _f32),
            pltpu.SemaphoreType.DMA,
            pltpu.SemaphoreType.DMA,
            pltpu.SemaphoreType.DMA,
        ],
    )
    def seg_sum(edges_h, r1d, out, *rest):
        idxs = rest[:_SD]
        bufs = rest[_SD:2 * _SD]
        zbuf, acc, sem_i, sem_l, sem_s = rest[2 * _SD:]
        cid = lax.axis_index("c")
        sid = lax.axis_index("s")
        wid = sid * _NC + cid
        c0 = wid * cpw

        zv = jnp.zeros((16,), _f32)

        def zrow(i, carry):
            for j in range(d // 16):
                zbuf[i, pl.ds(j * 16, 16)] = zv
            return carry

        lax.fori_loop(0, _ZB, zrow, 0, unroll=False)

        n_zmy = (N_NODES // _ZB - 1 - sid) // _NS + 1

        def zinit(i, carry):
            pltpu.sync_copy(zbuf, acc.at[pl.ds((i * _NS + sid) * _ZB, _ZB)])
            return carry

        lax.fori_loop(0, n_zmy, zinit, 0, unroll=False)
        plsc.subcore_barrier()

        def grp(j, carry):
            cbase = c0 + _SD * j
            di = [pltpu.async_copy(
                r1d.at[pl.ds((cbase + b) * _GC, _GC)], idxs[b],
                sem_i) for b in range(_SD)]
            dl = [pltpu.async_copy(
                edges_h.at[pl.ds(jnp.minimum(cbase + b, nch - 1) * _GC,
                                 _GC)], bufs[b],
                sem_l) for b in range(_SD)]
            for x in di:
                x.wait()
            for x in dl:
                x.wait()
            ds = [pltpu.async_copy(bufs[b], acc.at[idxs[b]], sem_s,
                                   add=True) for b in range(_SD)]
            for x in ds:
                x.wait()
            return carry

        lax.fori_loop(0, cpw // _SD, grp, 0, unroll=False)
        plsc.subcore_barrier()

        n_wmy = (_NZCH - 1 - sid) // _NS + 1

        def wout(i, carry):
            off = (i * _NS + sid) * _ZROWS
            pltpu.sync_copy(acc.at[pl.ds(off, _ZROWS)],
                            out.at[cid, pl.ds(off, _ZROWS)])
            return carry

        lax.fori_loop(0, n_wmy, wout, 0, unroll=False)

    return seg_sum


_BE = 2000
_BN = 2000


def _ln(h, s, o):
    m = jnp.mean(h, axis=-1, keepdims=True)
    v = jnp.mean((h - m) * (h - m), axis=-1, keepdims=True)
    return (h - m) * lax.rsqrt(v + 1e-5) * s + o


def _dot(x, w):
    return jnp.dot(x.astype(jnp.bfloat16), w, preferred_element_type=_f32)


def _row2(b, w=LATENT):
    return pl.BlockSpec((b, w), lambda i: (i, 0))


def _fix(a):
    return pl.BlockSpec(a.shape, lambda i: tuple(0 for _ in a.shape))


def _wcast(p):
    ws = [w.astype(jnp.bfloat16) for w in p["w"]]
    bs = [b.reshape(1, -1) for b in p["b"]]
    extra = []
    if "ln_s" in p:
        extra = [p["ln_s"].reshape(1, -1), p["ln_o"].reshape(1, -1)]
    return ws, bs, extra


def _prep_tables(last, wprep):
    def body(x_ref, w_ref, ta_out, tb_out):
        t = jnp.dot(x_ref[...], w_ref[...], preferred_element_type=_f32)
        ta_out[...] = -t
        tb_out[...] = t

    shp = jax.ShapeDtypeStruct((N_NODES, LATENT), _f32)
    return pl.pallas_call(
        body, grid=(N_NODES // _BN,),
        in_specs=[_row2(_BN, DIM), _fix(wprep)],
        out_specs=[_row2(_BN)] * 2,
        out_shape=[shp, shp],
    )(last, wprep)


def _edge_encoder(g, p, ne_rows):
    ws, bs, (lns, lno) = _wcast(p)
    w1 = p["w"][0]
    m = jnp.eye(LATENT, dtype=_f32).at[:DIM, :DIM].set(w1[:DIM, :DIM])
    m = m.astype(jnp.bfloat16)
    w4 = w1[DIM].reshape(1, LATENT)
    msk = jnp.zeros((1, LATENT), _f32).at[0, :DIM].set(1.0)

    def body(x_ref, m_, w4_, msk_, w2_, w3_, b1_, b2_, b3_, s_, o_, out):
        x = x_ref[...]
        xm = x * msk_[...]
        dist = jnp.sqrt(jnp.sum(xm * xm, axis=-1, keepdims=True))
        h = _dot(x, m_[...]) + dist * w4_[...] + b1_[...]
        h = jnp.maximum(h, 0.0)
        h = jnp.maximum(_dot(h, w2_[...]) + b2_[...], 0.0)
        h = _dot(h, w3_[...]) + b3_[...]
        out[...] = _ln(h, s_[...], o_[...])

    ne = ne_rows
    args = (g, m, w4, msk, ws[1], ws[2], bs[0], bs[1], bs[2], lns, lno)
    return pl.pallas_call(
        body, grid=(ne // _BE,),
        in_specs=[_row2(_BE)] + [_fix(a) for a in args[1:]],
        out_specs=_row2(_BE),
        out_shape=jax.ShapeDtypeStruct((ne, LATENT), _f32),
    )(*args)


def _node_encoder(wp_hi, wp_lo, p, wb, wc):
    ws, bs, (lns, lno) = _wcast(p)

    def body(a_ref, b_ref, w1_, w2_, w3_, b1_, b2_, b3_, s_, o_, wb_, wc_,
             n_out, pb_out, pc_out):
        vel = (a_ref[...] - b_ref[...]) * (1.0 / RADIUS)
        h = _dot(vel, w1_[...]) + b1_[...]
        h = jnp.maximum(h, 0.0)
        h = jnp.maximum(_dot(h, w2_[...]) + b2_[...], 0.0)
        h = _dot(h, w3_[...]) + b3_[...]
        n = _ln(h, s_[...], o_[...])
        n_out[...] = n
        pb_out[...] = _dot(n, wb_[...])
        pc_out[...] = _dot(n, wc_[...])

    args = (wp_hi, wp_lo, ws[0], ws[1], ws[2], bs[0], bs[1], bs[2], lns, lno,
            wb, wc)
    shp = jax.ShapeDtypeStruct((N_NODES, LATENT), _f32)
    return pl.pallas_call(
        body, grid=(N_NODES // _BN,),
        in_specs=[_row2(_BN, (SEQ_LEN - 1) * DIM)] * 2
        + [_fix(a) for a in args[2:]],
        out_specs=[_row2(_BN)] * 3,
        out_shape=[shp, shp, shp],
    )(*args)


def _edge_step(edges, gsum, p):
    ws, bs, (lns, lno) = _wcast(p)
    wa = ws[0][:LATENT]

    def body(e_ref, g_ref, wa_, w2_, w3_, b1_, b2_, b3_, s_, o_, out):
        x = e_ref[...]
        h = _dot(x, wa_[...]) + g_ref[...] + b1_[...]
        h = jnp.maximum(h, 0.0)
        h = jnp.maximum(_dot(h, w2_[...]) + b2_[...], 0.0)
        h = _dot(h, w3_[...]) + b3_[...]
        out[...] = x + _ln(h, s_[...], o_[...])

    ne = edges.shape[0]
    args = (edges, gsum, wa, ws[1], ws[2], bs[0], bs[1], bs[2], lns, lno)
    return pl.pallas_call(
        body, grid=(ne // _BE,),
        in_specs=[_row2(_BE), _row2(_BE)] + [_fix(a) for a in args[2:]],
        out_specs=_row2(_BE),
        out_shape=jax.ShapeDtypeStruct((ne, LATENT), _f32),
    )(*args)


def _node_step(nodes, aggs, p, wb, wc):
    ws, bs, (lns, lno) = _wcast(p)
    wna = ws[0][:LATENT]
    wnb = ws[0][LATENT:]
    with_proj = wb is not None
    na = len(aggs)

    def body(n_ref, *rest):
        a_refs = rest[:na]
        wna_, wnb_, w2_, w3_, b1_, b2_, b3_, s_, o_ = rest[na:na + 9]
        rest = rest[na + 9:]
        if with_proj:
            wb_, wc_, n_out, pb_out, pc_out = rest
        else:
            wb_ = wc_ = pb_out = pc_out = None
            (n_out,) = rest
        x = n_ref[...]
        agg = a_refs[0][...]
        for a in a_refs[1:]:
            agg = agg + a[...]
        h = _dot(x, wna_[...]) + _dot(agg, wnb_[...]) + b1_[...]
        h = jnp.maximum(h, 0.0)
        h = jnp.maximum(_dot(h, w2_[...]) + b2_[...], 0.0)
        h = _dot(h, w3_[...]) + b3_[...]
        n = x + _ln(h, s_[...], o_[...])
        n_out[...] = n
        if with_proj:
            pb_out[...] = _dot(n, wb_[...])
            pc_out[...] = _dot(n, wc_[...])

    args = [nodes] + list(aggs) + [wna, wnb, ws[1], ws[2], bs[0], bs[1],
                                   bs[2], lns, lno]
    if with_proj:
        args += [wb, wc]
    shp = jax.ShapeDtypeStruct((N_NODES, LATENT), _f32)
    n_out = 3 if with_proj else 1
    res = pl.pallas_call(
        body, grid=(N_NODES // _BN,),
        in_specs=[_row2(_BN)] * (1 + na) + [_fix(a) for a in args[1 + na:]],
        out_specs=[_row2(_BN)] * n_out,
        out_shape=[shp] * n_out,
    )(*args)
    return res if with_proj else (res[0], None, None)


def _decoder(nodes, intg, p, stdp, meanp):
    ws = [w.astype(jnp.bfloat16) for w in p["w"]]
    w3p = jnp.zeros((LATENT, LATENT), jnp.bfloat16).at[:, :DIM].set(ws[2])
    bs = [b.reshape(1, -1) for b in p["b"]]
    b3p = jnp.zeros((1, LATENT), _f32).at[0, :DIM].set(p["b"][2])

    def body(n_ref, i_ref, w1_, w2_, w3_, b1_, b2_, b3_, st_, mn_, out):
        h = jnp.maximum(_dot(n_ref[...], w1_[...]) + b1_[...], 0.0)
        h = jnp.maximum(_dot(h, w2_[...]) + b2_[...], 0.0)
        h = _dot(h, w3_[...]) + b3_[...]
        out[...] = i_ref[...] + h * st_[...] + mn_[...]

    args = (nodes, intg, ws[0], ws[1], w3p, bs[0], bs[1], b3p, stdp, meanp)
    return pl.pallas_call(
        body, grid=(N_NODES // _BN,),
        in_specs=[_row2(_BN), _row2(_BN)] + [_fix(a) for a in args[2:]],
        out_specs=_row2(_BN),
        out_shape=jax.ShapeDtypeStruct((N_NODES, LATENT), _f32),
    )(*args)


_NHALF = 2
_EH = N_EDGES // _NHALF


def kernel(world_position, params, senders, receivers):
    s32 = senders.astype(_i32)
    r32 = receivers.astype(_i32)
    ncp = _NW * (-(-(_EH // _GC) // _NW)) * _GC
    sh = [s32[h * _EH:(h + 1) * _EH] for h in range(_NHALF)]
    rh = [r32[h * _EH:(h + 1) * _EH] for h in range(_NHALF)]
    rhd = [jnp.pad(r, (0, ncp - _EH), constant_values=N_NODES) for r in rh]
    padidx = (jnp.arange(ncp - _EH, dtype=_i32) * 997) % N_NODES
    shg = [jnp.concatenate([s, padidx]) for s in sh]
    rhg = [jnp.concatenate([r, padidx]) for r in rh]
    wp = world_position.reshape(N_NODES, SEQ_LEN * DIM)
    last = wp[:, (SEQ_LEN - 1) * DIM:]
    prev = wp[:, (SEQ_LEN - 2) * DIM:(SEQ_LEN - 1) * DIM]

    w1e = params["enc_edge"]["w"][0]
    wprep = jnp.concatenate(
        [jnp.eye(DIM, dtype=_f32), w1e[:DIM, DIM:]], axis=1) * (1.0 / RADIUS)
    ta, tb = _prep_tables(last, wprep)
    gather_add = _make_gather_add(_EH)
    grel = [gather_add(ta, tb, shg[h], rhg[h]) for h in range(_NHALF)]

    edges = [_edge_encoder(g, params["enc_edge"], _EH) for g in grel]

    pe0 = params["proc"][0]["edge"]
    wb0 = pe0["w"][0][LATENT:2 * LATENT].astype(jnp.bfloat16)
    wc0 = pe0["w"][0][2 * LATENT:].astype(jnp.bfloat16)
    nodes, nsb, nsc = _node_encoder(
        wp[:, DIM:], wp[:, :(SEQ_LEN - 1) * DIM],
        params["enc_node"], wb0, wc0)

    seg_sum = _make_segment_sum(_EH)
    n_steps = len(params["proc"])
    for i in range(n_steps):
        gsum = [gather_add(nsb, nsc, shg[h], rhg[h]) for h in range(_NHALF)]
        edges = [_edge_step(edges[h], gsum[h], params["proc"][i]["edge"])
                 for h in range(_NHALF)]
        aggs = []
        for h in range(_NHALF):
            a = seg_sum(edges[h], rhd[h])
            aggs += [a[0], a[1]]
        if i + 1 < n_steps:
            pe = params["proc"][i + 1]["edge"]
            wb = pe["w"][0][LATENT:2 * LATENT].astype(jnp.bfloat16)
            wc = pe["w"][0][2 * LATENT:].astype(jnp.bfloat16)
        else:
            wb = wc = None
        nodes, nsb, nsc = _node_step(
            nodes, aggs, params["proc"][i]["node"], wb, wc)

    intg = jnp.zeros((N_NODES, LATENT), _f32).at[:, :DIM].set(2.0 * last - prev)
    stdp = jnp.zeros((1, LATENT), _f32).at[0, :DIM].set(params["target_std"])
    meanp = jnp.zeros((1, LATENT), _f32).at[0, :DIM].set(params["target_mean"])
    out = _decoder(nodes, intg, params["dec"], stdp, meanp)
    return out[:, :DIM]
